# baseline (speedup 1.0000x reference)
import functools
import math

import jax
import jax.numpy as jnp
from jax import lax
from jax.experimental import pallas as pl
from jax.experimental.pallas import tpu as pltpu
from jax.experimental.pallas import tpu_sc as plsc

DIM = 64
LANES = 16
CHUNK = 128
NBUF = 2
SCALE = math.sqrt(DIM)


def _process_chunk(src, dst, b, idx_v, j):
    nseg = CHUNK // LANES
    rowvs = [jnp.arange(LANES, dtype=jnp.int32) + (s * LANES) for s in range(nseg)]
    offvs = []
    for s in range(nseg):
        iv = idx_v[j, pl.ds(s * LANES, LANES)]
        offvs.append((iv & 1) * jnp.int32(DIM))

    def per_d(d, carry):
        dcol = jnp.full((LANES,), d, dtype=jnp.int32)
        vals = []
        for s in range(nseg):
            v = plsc.load_gather(src.at[b], [rowvs[s], offvs[s] + d])
            vals.append(v * SCALE)
        for s in range(nseg):
            plsc.store_scatter(dst.at[b], [rowvs[s], dcol], vals[s])
        return carry

    lax.fori_loop(0, DIM, per_d, 0, unroll=2)


@functools.lru_cache(maxsize=None)
def _make_gather(NW, NC, n_chunks, b_per_w, B):
    mesh = plsc.VectorSubcoreMesh(core_axis_name="c", subcore_axis_name="s")

    @functools.partial(
        pl.kernel,
        out_type=jax.ShapeDtypeStruct((B, DIM), jnp.float32),
        mesh=mesh,
        compiler_params=pltpu.CompilerParams(
            use_tc_tiling_on_sc=True, needs_layout_passes=False
        ),
        scratch_types=[
            pltpu.VMEM((n_chunks, CHUNK), jnp.int32),
            pltpu.VMEM((n_chunks, CHUNK), jnp.int32),
            pltpu.VMEM((NBUF, CHUNK, 2 * DIM), jnp.float32),
            pltpu.VMEM((NBUF, CHUNK, DIM), jnp.float32),
            pltpu.SemaphoreType.DMA,
            pltpu.SemaphoreType.DMA,
            pltpu.SemaphoreType.DMA,
            pltpu.SemaphoreType.DMA,
        ],
    )
    def body(idx_hbm, table_hbm, out_hbm, idx_v, lid_v, bufg, bufo,
             sg0, sg1, so0, so1):
        semg = (sg0, sg1)
        semo = (so0, so1)
        wid = lax.axis_index("s") * NC + lax.axis_index("c")
        base = wid * b_per_w
        pltpu.sync_copy(idx_hbm.at[wid], idx_v)

        def shift_row(r, carry):
            for s in range(CHUNK // LANES):
                sl = pl.ds(s * LANES, LANES)
                lid_v[r, sl] = lax.shift_right_logical(idx_v[r, sl], 1)
            return carry

        lax.fori_loop(0, n_chunks, shift_row, 0, unroll=4)

        def g_start(j, b):
            pltpu.async_copy(table_hbm.at[lid_v.at[j]], bufg.at[b], semg[b])

        def g_wait(j, b):
            pltpu.make_async_copy(
                table_hbm.at[lid_v.at[j]], bufg.at[b], semg[b]
            ).wait()

        def o_start(j, b):
            pltpu.async_copy(
                bufo.at[b], out_hbm.at[pl.ds(base + j * CHUNK, CHUNK)], semo[b]
            )

        def o_wait(j, b):
            pltpu.make_async_copy(
                bufo.at[b], out_hbm.at[pl.ds(base + j * CHUNK, CHUNK)], semo[b]
            ).wait()

        for b in range(NBUF):
            g_start(b, b)

        for j in range(NBUF):
            b = j
            g_wait(j, b)
            _process_chunk(bufg, bufo, b, idx_v, j)
            g_start(j + NBUF, b)
            o_start(j, b)

        def outer(i, carry):
            for b in range(NBUF):
                j = i * NBUF + b
                g_wait(j, b)
                o_wait(j - NBUF, b)
                _process_chunk(bufg, bufo, b, idx_v, j)
                g_start(j + NBUF, b)
                o_start(j, b)
            return carry

        lax.fori_loop(1, n_chunks // NBUF - 1, outer, 0)

        for b in range(NBUF):
            j = n_chunks - NBUF + b
            g_wait(j, b)
            o_wait(j - NBUF, b)
            _process_chunk(bufg, bufo, b, idx_v, j)
            o_start(j, b)
        for b in range(NBUF):
            o_wait(n_chunks - NBUF + b, b)

    return body


def kernel(x, table):
    batch, seq = x.shape
    B = batch * seq
    info = plsc.get_sparse_core_info()
    NC, NS = info.num_cores, info.num_subcores
    NW = NC * NS
    b_per_w = B // NW
    n_chunks = b_per_w // CHUNK
    idx = x.reshape(NW, n_chunks, CHUNK).astype(jnp.int32)
    table2 = table.reshape(table.shape[0] // 2, 2 * DIM)
    out = _make_gather(NW, NC, n_chunks, b_per_w, B)(idx, table2)
    return out.reshape(batch, seq, DIM)


# speedup vs baseline: 2.5987x; 2.5987x over previous
import functools
import math

import jax
import jax.numpy as jnp
from jax import lax
from jax.experimental import pallas as pl
from jax.experimental.pallas import tpu as pltpu
from jax.experimental.pallas import tpu_sc as plsc

DIM = 64
LANES = 16
CHUNK = 128
NBUF = 2
SCALE = math.sqrt(DIM)

ROWS_PER_IT = 4


def _scale_chunk(src, dst, b):
    def rows(i, carry):
        r0 = i * ROWS_PER_IT
        vals = []
        for dr in range(ROWS_PER_IT):
            for k in range(DIM // LANES):
                sl = pl.ds(k * LANES, LANES)
                vals.append((dr, sl, src[b, r0 + dr, sl] * SCALE))
        for dr, sl, v in vals:
            dst[b, r0 + dr, sl] = v
        return carry

    lax.fori_loop(0, CHUNK // ROWS_PER_IT, rows, 0)


@functools.lru_cache(maxsize=None)
def _make_gather(NW, NC, n_chunks, b_per_w, B):
    mesh = plsc.VectorSubcoreMesh(core_axis_name="c", subcore_axis_name="s")

    @functools.partial(
        pl.kernel,
        out_type=jax.ShapeDtypeStruct((B, DIM), jnp.float32),
        mesh=mesh,
        compiler_params=pltpu.CompilerParams(
            use_tc_tiling_on_sc=True, needs_layout_passes=False
        ),
        scratch_types=[
            pltpu.VMEM((n_chunks, CHUNK), jnp.int32),
            pltpu.VMEM((NBUF, CHUNK, 2 * DIM), jnp.float32),
            pltpu.VMEM((NBUF, CHUNK, DIM), jnp.float32),
            pltpu.SemaphoreType.DMA,
            pltpu.SemaphoreType.DMA,
            pltpu.SemaphoreType.DMA,
            pltpu.SemaphoreType.DMA,
        ],
    )
    def body(idx_hbm, table_hbm, out_hbm, idx_v, bufg, bufo, sg0, sg1, so0, so1):
        semg = (sg0, sg1)
        semo = (so0, so1)
        wid = lax.axis_index("s") * NC + lax.axis_index("c")
        base = wid * b_per_w
        pltpu.sync_copy(idx_hbm.at[wid], idx_v)

        def g_start(j, b):
            pltpu.async_copy(table_hbm.at[idx_v.at[j]], bufg.at[b], semg[b])

        def g_wait(j, b):
            pltpu.make_async_copy(
                table_hbm.at[idx_v.at[j]], bufg.at[b], semg[b]
            ).wait()

        def o_start(j, b):
            pltpu.async_copy(
                bufo.at[b], out_hbm.at[pl.ds(base + j * CHUNK, CHUNK)], semo[b]
            )

        def o_wait(j, b):
            pltpu.make_async_copy(
                bufo.at[b], out_hbm.at[pl.ds(base + j * CHUNK, CHUNK)], semo[b]
            ).wait()

        for b in range(NBUF):
            g_start(b, b)

        for j in range(NBUF):
            b = j
            g_wait(j, b)
            _scale_chunk(bufg, bufo, b)
            g_start(j + NBUF, b)
            o_start(j, b)

        def outer(i, carry):
            for b in range(NBUF):
                j = i * NBUF + b
                g_wait(j, b)
                o_wait(j - NBUF, b)
                _scale_chunk(bufg, bufo, b)
                g_start(j + NBUF, b)
                o_start(j, b)
            return carry

        lax.fori_loop(1, n_chunks // NBUF - 1, outer, 0)

        for b in range(NBUF):
            j = n_chunks - NBUF + b
            g_wait(j, b)
            o_wait(j - NBUF, b)
            _scale_chunk(bufg, bufo, b)
            o_start(j, b)
        for b in range(NBUF):
            o_wait(n_chunks - NBUF + b, b)

    return body


def kernel(x, table):
    batch, seq = x.shape
    B = batch * seq
    info = plsc.get_sparse_core_info()
    NC, NS = info.num_cores, info.num_subcores
    NW = NC * NS
    b_per_w = B // NW
    n_chunks = b_per_w // CHUNK
    idx = x.reshape(NW, n_chunks, CHUNK).astype(jnp.int32)
    table2 = jnp.pad(table, ((0, 0), (0, DIM)))
    out = _make_gather(NW, NC, n_chunks, b_per_w, B)(idx, table2)
    return out.reshape(batch, seq, DIM)
